# baseline (speedup 1.0000x reference)
import jax
import jax.numpy as jnp
from jax import lax
from jax._src.pallas import mpmd as _mpmd
from jax.experimental import pallas as pl
from jax.experimental.pallas import tpu as pltpu
from jax.experimental.pallas import tpu_sc as plsc

BUF_SIZE = 1000000
FEAT = 32
BATCH = 16384
NUM_CORES = 2
NUM_SUBCORES = 16
L = 16
NW = NUM_CORES * NUM_SUBCORES
RPW = BUF_SIZE // NW
NCHUNK = BATCH // L
CCAP = BATCH + 192
DUMP = CCAP - 1
DMA_B = 128
NSTAT = 6

_mesh = plsc.VectorSubcoreMesh(
    core_axis_name="c",
    subcore_axis_name="s",
    num_cores=NUM_CORES,
    num_subcores=NUM_SUBCORES,
)


def _sc_body(
    x_hbm,
    y_hbm,
    tvec_hbm,
    idx_hbm,
    data_in,
    tgt_in,
    task_in,
    data_ref,
    tgt_ref,
    task_ref,
    idx_v,
    jl,
    fj,
    fi,
    table,
    jrow,
    irow,
    rows,
    yrow,
    trow,
    sem_in,
    sem_out,
):
    wid = lax.axis_index("s") * NUM_CORES + lax.axis_index("c")
    base = wid * RPW
    iota = lax.iota(jnp.int32, L)

    pltpu.sync_copy(idx_hbm, idx_v)
    pltpu.sync_copy(tvec_hbm, trow.at[pl.ds(0, L)])
    tv = trow[pl.ds(0, L)]
    for k in range(1, DMA_B // L):
        trow[pl.ds(k * L, L)] = tv

    def scan_body(c, cnt):
        v = idx_v[pl.ds(c * L, L)]
        loc = v - base
        m = (loc >= 0) & (loc < RPW)
        mi = jnp.where(m, 1, 0)
        pc = plsc.cumsum(mi)
        dest = jnp.where(m, cnt + pc - 1, DUMP)
        plsc.store_scatter(jl, [dest], iota + c * L)
        return cnt + jnp.sum(mi)

    cnt = lax.fori_loop(0, NCHUNK, scan_body, jnp.int32(0), unroll=4)
    nc = (cnt + L - 1) // L

    def dedup_mark(p, carry):
        lanep = iota + p * L
        lv = lanep < cnt
        jv = jl[pl.ds(p * L, L)]
        jvs = jnp.clip(jv, 0, BATCH - 1)
        av = plsc.load_gather(idx_v, [jvs])
        loc = av - base
        locs = jnp.where(lv, loc, -1 - iota)
        kill = jnp.zeros((L,), jnp.bool_)
        for s in range(1, L):
            perm = jnp.minimum(iota + s, L - 1)
            shifted = jnp.take_along_axis(
                locs, perm, axis=0, mode="promise_in_bounds"
            )
            kill = kill | ((locs == shifted) & (iota < (L - s)))
        keep = lv & ~kill
        dest = jnp.where(keep, locs, RPW)
        plsc.store_scatter(table, [dest], lanep)
        return carry

    lax.fori_loop(0, nc, dedup_mark, jnp.int32(0))

    def dedup_select(p, wcnt):
        lanep = iota + p * L
        lv = lanep < cnt
        jv = jl[pl.ds(p * L, L)]
        jvs = jnp.clip(jv, 0, BATCH - 1)
        av = plsc.load_gather(idx_v, [jvs])
        loc = jnp.where(lv, av - base, RPW)
        locs = jnp.clip(loc, 0, RPW)
        w = plsc.load_gather(table, [locs])
        win = lv & (w == lanep)
        wi = jnp.where(win, 1, 0)
        pc = plsc.cumsum(wi)
        dest = jnp.where(win, wcnt + pc - 1, DUMP)
        plsc.store_scatter(fj, [dest], jv)
        plsc.store_scatter(fi, [dest], av)
        return wcnt + jnp.sum(wi)

    wcnt = lax.fori_loop(0, nc, dedup_select, jnp.int32(0))

    @pl.when(wcnt > 0)
    def _():
        lastpos = jnp.full((L,), wcnt - 1, dtype=jnp.int32)
        lastj = plsc.load_gather(fj, [lastpos])
        lasti = plsc.load_gather(fi, [lastpos])
        for k in range(DMA_B // L):
            fj[pl.ds(wcnt + k * L, L)] = lastj
            fi[pl.ds(wcnt + k * L, L)] = lasti
        nchunks = (wcnt + DMA_B - 1) // DMA_B

        for c in range(NSTAT):

            @pl.when(c < nchunks)
            def _(c=c):
                for k in range(DMA_B // L):
                    jrow[c, pl.ds(k * L, L)] = fj[pl.ds(c * DMA_B + k * L, L)]
                    irow[c, pl.ds(k * L, L)] = fi[pl.ds(c * DMA_B + k * L, L)]
                pltpu.async_copy(x_hbm.at[jrow.at[c]], rows.at[c], sem_in)
                pltpu.async_copy(y_hbm.at[jrow.at[c]], yrow.at[c], sem_in)

        for c in range(NSTAT):

            @pl.when(c < nchunks)
            def _(c=c):
                pltpu.make_async_copy(
                    x_hbm.at[jrow.at[c]], rows.at[c], sem_in
                ).wait()
                pltpu.make_async_copy(
                    y_hbm.at[jrow.at[c]], yrow.at[c], sem_in
                ).wait()
                pltpu.async_copy(rows.at[c], data_ref.at[irow.at[c]], sem_out)
                pltpu.async_copy(yrow.at[c], tgt_ref.at[irow.at[c]], sem_out)
                pltpu.async_copy(trow, task_ref.at[irow.at[c]], sem_out)

        for c in range(NSTAT):

            @pl.when(c < nchunks)
            def _(c=c):
                pltpu.make_async_copy(
                    rows.at[c], data_ref.at[irow.at[c]], sem_out
                ).wait()
                pltpu.make_async_copy(
                    yrow.at[c], tgt_ref.at[irow.at[c]], sem_out
                ).wait()
                pltpu.make_async_copy(
                    trow, task_ref.at[irow.at[c]], sem_out
                ).wait()

        @pl.when(nchunks > NSTAT)
        def _():
            def dma_body(c, carry):
                def cds(k):
                    return pl.ds(c * DMA_B + k * L, L)

                for k in range(DMA_B // L):
                    jrow[0, pl.ds(k * L, L)] = fj[cds(k)]
                    irow[0, pl.ds(k * L, L)] = fi[cds(k)]
                gx = pltpu.async_copy(x_hbm.at[jrow.at[0]], rows.at[0], sem_in)
                gy = pltpu.async_copy(y_hbm.at[jrow.at[0]], yrow.at[0], sem_in)
                gx.wait()
                gy.wait()
                sx = pltpu.async_copy(
                    rows.at[0], data_ref.at[irow.at[0]], sem_out
                )
                sy = pltpu.async_copy(
                    yrow.at[0], tgt_ref.at[irow.at[0]], sem_out
                )
                st = pltpu.async_copy(trow, task_ref.at[irow.at[0]], sem_out)
                sx.wait()
                sy.wait()
                st.wait()
                return carry

            lax.fori_loop(NSTAT, nchunks, dma_body, jnp.int32(0))


_sc_update = _mpmd._mpmd_map(
    [(_mesh, _sc_body)],
    out_types=(
        jax.ShapeDtypeStruct((BUF_SIZE, FEAT), jnp.float32),
        jax.ShapeDtypeStruct((BUF_SIZE,), jnp.int32),
        jax.ShapeDtypeStruct((BUF_SIZE,), jnp.int32),
    ),
    input_output_aliases={4: 0, 5: 1, 6: 2},
    scratch_types=[
        pltpu.VMEM((BATCH,), jnp.int32),
        pltpu.VMEM((CCAP,), jnp.int32),
        pltpu.VMEM((CCAP,), jnp.int32),
        pltpu.VMEM((CCAP,), jnp.int32),
        pltpu.VMEM((RPW + 1,), jnp.int32),
        pltpu.VMEM((NSTAT, DMA_B), jnp.int32),
        pltpu.VMEM((NSTAT, DMA_B), jnp.int32),
        pltpu.VMEM((NSTAT, DMA_B, FEAT), jnp.float32),
        pltpu.VMEM((NSTAT, DMA_B), jnp.int32),
        pltpu.VMEM((DMA_B,), jnp.int32),
        pltpu.SemaphoreType.DMA,
        pltpu.SemaphoreType.DMA,
    ],
    compiler_params=pltpu.CompilerParams(
        needs_layout_passes=False, use_tc_tiling_on_sc=False
    ),
)


def kernel(x, y, task, idx, buf_data, buf_targets, buf_tasks):
    task_vec = jnp.full((L,), task, dtype=jnp.int32)
    return _sc_update(x, y, task_vec, idx, buf_data, buf_targets, buf_tasks)


# speedup vs baseline: 1.0124x; 1.0124x over previous
import jax
import jax.numpy as jnp
from jax import lax
from jax._src.pallas import mpmd as _mpmd
from jax.experimental import pallas as pl
from jax.experimental.pallas import tpu as pltpu
from jax.experimental.pallas import tpu_sc as plsc

BUF_SIZE = 1000000
FEAT = 32
BATCH = 16384
NUM_CORES = 2
NUM_SUBCORES = 16
L = 16
NW = NUM_CORES * NUM_SUBCORES
RPW = BUF_SIZE // NW
NCHUNK = BATCH // L
CCAP = BATCH + 192
DUMP = CCAP - 1
DMA_B = 128
NSTAT = 6

_mesh = plsc.VectorSubcoreMesh(
    core_axis_name="c",
    subcore_axis_name="s",
    num_cores=NUM_CORES,
    num_subcores=NUM_SUBCORES,
)

_params = pltpu.CompilerParams(
    needs_layout_passes=False, use_tc_tiling_on_sc=False
)


def _worker_id():
    return lax.axis_index("s") * NUM_CORES + lax.axis_index("c")


def _lists_body(idx_hbm, fjh, fih, cnth, idx_v, jl, fj, fi, table, sem_out):
    wid = _worker_id()
    base = wid * RPW
    iota = lax.iota(jnp.int32, L)

    pltpu.sync_copy(idx_hbm, idx_v)

    def scan_body(c, cnt):
        v = idx_v[pl.ds(c * L, L)]
        loc = v - base
        m = (loc >= 0) & (loc < RPW)
        mi = jnp.where(m, 1, 0)
        pc = plsc.cumsum(mi)
        dest = jnp.where(m, cnt + pc - 1, DUMP)
        plsc.store_scatter(jl, [dest], iota + c * L)
        return cnt + jnp.sum(mi)

    cnt = lax.fori_loop(0, NCHUNK, scan_body, jnp.int32(0), unroll=4)
    nc = (cnt + L - 1) // L

    def dedup_mark(p, carry):
        lanep = iota + p * L
        lv = lanep < cnt
        jv = jl[pl.ds(p * L, L)]
        jvs = jnp.clip(jv, 0, BATCH - 1)
        av = plsc.load_gather(idx_v, [jvs])
        loc = av - base
        locs = jnp.where(lv, loc, -1 - iota)
        kill = jnp.zeros((L,), jnp.bool_)
        for s in range(1, L):
            perm = jnp.minimum(iota + s, L - 1)
            shifted = jnp.take_along_axis(
                locs, perm, axis=0, mode="promise_in_bounds"
            )
            kill = kill | ((locs == shifted) & (iota < (L - s)))
        keep = lv & ~kill
        dest = jnp.where(keep, locs, RPW)
        plsc.store_scatter(table, [dest], lanep)
        return carry

    lax.fori_loop(0, nc, dedup_mark, jnp.int32(0))

    def dedup_select(p, wcnt):
        lanep = iota + p * L
        lv = lanep < cnt
        jv = jl[pl.ds(p * L, L)]
        jvs = jnp.clip(jv, 0, BATCH - 1)
        av = plsc.load_gather(idx_v, [jvs])
        loc = jnp.where(lv, av - base, RPW)
        locs = jnp.clip(loc, 0, RPW)
        w = plsc.load_gather(table, [locs])
        win = lv & (w == lanep)
        wi = jnp.where(win, 1, 0)
        pc = plsc.cumsum(wi)
        dest = jnp.where(win, wcnt + pc - 1, DUMP)
        plsc.store_scatter(fj, [dest], jv)
        plsc.store_scatter(fi, [dest], av)
        return wcnt + jnp.sum(wi)

    wcnt = lax.fori_loop(0, nc, dedup_select, jnp.int32(0))

    @pl.when(wcnt > 0)
    def _():
        lastpos = jnp.full((L,), wcnt - 1, dtype=jnp.int32)
        lastj = plsc.load_gather(fj, [lastpos])
        lasti = plsc.load_gather(fi, [lastpos])
        for k in range(DMA_B // L):
            fj[pl.ds(wcnt + k * L, L)] = lastj
            fi[pl.ds(wcnt + k * L, L)] = lasti

    jl[pl.ds(0, L)] = jnp.full((L,), wcnt, dtype=jnp.int32)
    c1 = pltpu.async_copy(fj, fjh.at[wid], sem_out)
    c2 = pltpu.async_copy(fi, fih.at[wid], sem_out)
    c3 = pltpu.async_copy(jl.at[pl.ds(0, L)], cnth.at[pl.ds(wid * L, L)], sem_out)
    c1.wait()
    c2.wait()
    c3.wait()


_sc_lists = _mpmd._mpmd_map(
    [(_mesh, _lists_body)],
    out_types=(
        jax.ShapeDtypeStruct((NW, CCAP), jnp.int32),
        jax.ShapeDtypeStruct((NW, CCAP), jnp.int32),
        jax.ShapeDtypeStruct((NW * L,), jnp.int32),
    ),
    scratch_types=[
        pltpu.VMEM((BATCH,), jnp.int32),
        pltpu.VMEM((CCAP,), jnp.int32),
        pltpu.VMEM((CCAP,), jnp.int32),
        pltpu.VMEM((CCAP,), jnp.int32),
        pltpu.VMEM((RPW + 1,), jnp.int32),
        pltpu.SemaphoreType.DMA,
    ],
    compiler_params=_params,
)


def _apply_body(
    x_hbm,
    y_hbm,
    tvec_hbm,
    fjh,
    fih,
    cnth,
    data_in,
    tgt_in,
    task_in,
    data_ref,
    tgt_ref,
    task_ref,
    fj,
    fi,
    cv,
    jrow,
    irow,
    rows,
    yrow,
    trow,
    sem_in,
    sem_out,
):
    wid = _worker_id()

    g1 = pltpu.async_copy(fjh.at[wid], fj, sem_in)
    g2 = pltpu.async_copy(fih.at[wid], fi, sem_in)
    g3 = pltpu.async_copy(cnth.at[pl.ds(wid * L, L)], cv, sem_in)
    pltpu.sync_copy(tvec_hbm, trow.at[pl.ds(0, L)])
    tv = trow[pl.ds(0, L)]
    for k in range(1, DMA_B // L):
        trow[pl.ds(k * L, L)] = tv
    g1.wait()
    g2.wait()
    g3.wait()
    wcnt = jnp.max(cv[pl.ds(0, L)])

    @pl.when(wcnt > 0)
    def _():
        nchunks = (wcnt + DMA_B - 1) // DMA_B

        for c in range(NSTAT):

            @pl.when(c < nchunks)
            def _(c=c):
                for k in range(DMA_B // L):
                    jrow[c, pl.ds(k * L, L)] = fj[pl.ds(c * DMA_B + k * L, L)]
                    irow[c, pl.ds(k * L, L)] = fi[pl.ds(c * DMA_B + k * L, L)]
                pltpu.async_copy(x_hbm.at[jrow.at[c]], rows.at[c], sem_in)
                pltpu.async_copy(y_hbm.at[jrow.at[c]], yrow.at[c], sem_in)

        for c in range(NSTAT):

            @pl.when(c < nchunks)
            def _(c=c):
                pltpu.make_async_copy(
                    x_hbm.at[jrow.at[c]], rows.at[c], sem_in
                ).wait()
                pltpu.make_async_copy(
                    y_hbm.at[jrow.at[c]], yrow.at[c], sem_in
                ).wait()
                pltpu.async_copy(rows.at[c], data_ref.at[irow.at[c]], sem_out)
                pltpu.async_copy(yrow.at[c], tgt_ref.at[irow.at[c]], sem_out)
                pltpu.async_copy(trow, task_ref.at[irow.at[c]], sem_out)

        for c in range(NSTAT):

            @pl.when(c < nchunks)
            def _(c=c):
                pltpu.make_async_copy(
                    rows.at[c], data_ref.at[irow.at[c]], sem_out
                ).wait()
                pltpu.make_async_copy(
                    yrow.at[c], tgt_ref.at[irow.at[c]], sem_out
                ).wait()
                pltpu.make_async_copy(
                    trow, task_ref.at[irow.at[c]], sem_out
                ).wait()

        @pl.when(nchunks > NSTAT)
        def _():
            def dma_body(c, carry):
                for k in range(DMA_B // L):
                    jrow[0, pl.ds(k * L, L)] = fj[pl.ds(c * DMA_B + k * L, L)]
                    irow[0, pl.ds(k * L, L)] = fi[pl.ds(c * DMA_B + k * L, L)]
                gx = pltpu.async_copy(x_hbm.at[jrow.at[0]], rows.at[0], sem_in)
                gy = pltpu.async_copy(y_hbm.at[jrow.at[0]], yrow.at[0], sem_in)
                gx.wait()
                gy.wait()
                sx = pltpu.async_copy(
                    rows.at[0], data_ref.at[irow.at[0]], sem_out
                )
                sy = pltpu.async_copy(
                    yrow.at[0], tgt_ref.at[irow.at[0]], sem_out
                )
                st = pltpu.async_copy(trow, task_ref.at[irow.at[0]], sem_out)
                sx.wait()
                sy.wait()
                st.wait()
                return carry

            lax.fori_loop(NSTAT, nchunks, dma_body, jnp.int32(0))


_sc_apply = _mpmd._mpmd_map(
    [(_mesh, _apply_body)],
    out_types=(
        jax.ShapeDtypeStruct((BUF_SIZE, FEAT), jnp.float32),
        jax.ShapeDtypeStruct((BUF_SIZE,), jnp.int32),
        jax.ShapeDtypeStruct((BUF_SIZE,), jnp.int32),
    ),
    input_output_aliases={6: 0, 7: 1, 8: 2},
    scratch_types=[
        pltpu.VMEM((CCAP,), jnp.int32),
        pltpu.VMEM((CCAP,), jnp.int32),
        pltpu.VMEM((L,), jnp.int32),
        pltpu.VMEM((NSTAT, DMA_B), jnp.int32),
        pltpu.VMEM((NSTAT, DMA_B), jnp.int32),
        pltpu.VMEM((NSTAT, DMA_B, FEAT), jnp.float32),
        pltpu.VMEM((NSTAT, DMA_B), jnp.int32),
        pltpu.VMEM((DMA_B,), jnp.int32),
        pltpu.SemaphoreType.DMA,
        pltpu.SemaphoreType.DMA,
    ],
    compiler_params=_params,
)


def kernel(x, y, task, idx, buf_data, buf_targets, buf_tasks):
    task_vec = jnp.full((L,), task, dtype=jnp.int32)
    fjh, fih, cnth = _sc_lists(idx)
    return _sc_apply(
        x, y, task_vec, fjh, fih, cnth, buf_data, buf_targets, buf_tasks
    )
